# baseline (speedup 1.0000x reference)
import functools

import jax
import jax.numpy as jnp
from jax.experimental import pallas as pl
from jax.experimental.pallas import tpu as pltpu

_E = 64
_W = 128
_TOPK = 8
_TB = 512


def _moe_body(x_ref, w1_ref, w2_ref, c_ref, out_ref):
    e = pl.program_id(1)
    h = jnp.dot(x_ref[...], w1_ref[0], preferred_element_type=jnp.float32)
    h = jnp.square(jnp.maximum(h, 0.0))
    h = h * c_ref[0, 0, 0, :][:, None]
    y = jnp.dot(h, w2_ref[0], preferred_element_type=jnp.float32)

    @pl.when(e == 0)
    def _():
        out_ref[...] = y

    @pl.when(e != 0)
    def _():
        out_ref[...] += y


def kernel(x, router_w, w1, w2):
    b, s, d = x.shape
    T = b * s
    x_flat = x.reshape(T, d)

    router_logits = x_flat @ router_w.T
    router_probs = jax.nn.sigmoid(router_logits)
    top_k_weights, selected_experts = jax.lax.top_k(router_probs, _TOPK)
    top_k_weights = top_k_weights / (top_k_weights.sum(axis=-1, keepdims=True) + 1e-20)
    combine = jnp.zeros((T, _E), dtype=x.dtype).at[
        jnp.arange(T)[:, None], selected_experts].add(top_k_weights)

    n_tb = T // _TB
    combine_r = combine.reshape(n_tb, _TB, _E).transpose(0, 2, 1)
    combine_r = combine_r.reshape(n_tb, _E, 1, _TB)
    w1_r = w1.reshape(d, _E, _W).transpose(1, 0, 2)
    w2_r = w2.reshape(_E, _W, d)

    out_flat = pl.pallas_call(
        _moe_body,
        grid=(n_tb, _E),
        in_specs=[
            pl.BlockSpec((_TB, d), lambda t, e: (t, 0)),
            pl.BlockSpec((1, d, _W), lambda t, e: (e, 0, 0)),
            pl.BlockSpec((1, _W, d), lambda t, e: (e, 0, 0)),
            pl.BlockSpec((1, 1, 1, _TB), lambda t, e: (t, e, 0, 0)),
        ],
        out_specs=pl.BlockSpec((_TB, d), lambda t, e: (t, 0)),
        out_shape=jax.ShapeDtypeStruct((T, d), jnp.float32),
        compiler_params=pltpu.CompilerParams(
            dimension_semantics=("parallel", "arbitrary")),
    )(x_flat, w1_r.reshape(_E, d, _W), w2_r, combine_r)

    output = out_flat.reshape(b, s, d)

    router_z_loss = jnp.mean(jnp.square(jax.nn.logsumexp(router_logits, axis=-1)))
    tokens_per_expert = jnp.bincount(selected_experts.reshape(-1), length=_E)
    f_i = tokens_per_expert.astype(jnp.float32) / tokens_per_expert.sum()
    p_i = router_probs.mean(axis=0)
    load_balance_loss = float(_E) * (f_i @ p_i)
    compute_loss = jnp.mean(router_probs @ jnp.ones((_E,), jnp.float32))
    return (output, router_z_loss, load_balance_loss, compute_loss, f_i)


# speedup vs baseline: 1.1888x; 1.1888x over previous
import functools

import jax
import jax.numpy as jnp
from jax.experimental import pallas as pl
from jax.experimental.pallas import tpu as pltpu

_E = 64
_W = 128
_TOPK = 8
_TB = 512


def _moe_body(x_ref, w1_ref, w2_ref, c_ref, out_ref):
    e = pl.program_id(0)
    h = jnp.dot(x_ref[...], w1_ref[0], preferred_element_type=jnp.float32)
    h = jnp.square(jnp.maximum(h, 0.0))
    h = h * c_ref[0, 0, :][:, None]
    y = jnp.dot(h.astype(jnp.bfloat16), w2_ref[0],
                preferred_element_type=jnp.float32)

    @pl.when(e == 0)
    def _():
        out_ref[...] = y

    @pl.when(e != 0)
    def _():
        out_ref[...] += y


def kernel(x, router_w, w1, w2):
    b, s, d = x.shape
    T = b * s
    x_flat = x.reshape(T, d)

    router_logits = x_flat @ router_w.T
    router_probs = jax.nn.sigmoid(router_logits)
    top_k_weights, selected_experts = jax.lax.top_k(router_probs, _TOPK)
    top_k_weights = top_k_weights / (top_k_weights.sum(axis=-1, keepdims=True) + 1e-20)
    combine = jnp.zeros((T, _E), dtype=x.dtype).at[
        jnp.arange(T)[:, None], selected_experts].add(top_k_weights)

    combine_r = combine.T.reshape(_E, 1, T)
    w1_r = w1.reshape(d, _E, _W).transpose(1, 0, 2).astype(jnp.bfloat16)
    w2_r = w2.reshape(_E, _W, d).astype(jnp.bfloat16)

    out_flat = pl.pallas_call(
        _moe_body,
        grid=(_E,),
        in_specs=[
            pl.BlockSpec((T, d), lambda e: (0, 0)),
            pl.BlockSpec((1, d, _W), lambda e: (e, 0, 0)),
            pl.BlockSpec((1, _W, d), lambda e: (e, 0, 0)),
            pl.BlockSpec((1, 1, T), lambda e: (e, 0, 0)),
        ],
        out_specs=pl.BlockSpec((T, d), lambda e: (0, 0)),
        out_shape=jax.ShapeDtypeStruct((T, d), jnp.float32),
        compiler_params=pltpu.CompilerParams(
            dimension_semantics=("arbitrary",)),
    )(x_flat.astype(jnp.bfloat16), w1_r, w2_r, combine_r)

    output = out_flat.reshape(b, s, d)

    router_z_loss = jnp.mean(jnp.square(jax.nn.logsumexp(router_logits, axis=-1)))
    tokens_per_expert = jnp.bincount(selected_experts.reshape(-1), length=_E)
    f_i = tokens_per_expert.astype(jnp.float32) / tokens_per_expert.sum()
    p_i = router_probs.mean(axis=0)
    load_balance_loss = float(_E) * (f_i @ p_i)
    compute_loss = jnp.mean(router_probs @ jnp.ones((_E,), jnp.float32))
    return (output, router_z_loss, load_balance_loss, compute_loss, f_i)


# speedup vs baseline: 1.4349x; 1.2070x over previous
import functools

import jax
import jax.numpy as jnp
from jax.experimental import pallas as pl
from jax.experimental.pallas import tpu as pltpu

_E = 64
_W = 128
_TOPK = 8
_TB = 512


def _moe_body(x_ref, w1_ref, w2_ref, c_ref, out_ref):
    e = pl.program_id(0)
    h = jnp.dot(x_ref[...], w1_ref[0], preferred_element_type=jnp.float32)
    h = jnp.square(jnp.maximum(h, 0.0))
    h = h * c_ref[0, 0, :][:, None]
    y = jnp.dot(h.astype(jnp.bfloat16), w2_ref[0],
                preferred_element_type=jnp.float32)

    @pl.when(e == 0)
    def _():
        out_ref[...] = y

    @pl.when(e != 0)
    def _():
        out_ref[...] += y


def kernel(x, router_w, w1, w2):
    b, s, d = x.shape
    T = b * s
    x_flat = x.reshape(T, d)

    router_logits = x_flat @ router_w.T
    router_probs = jax.nn.sigmoid(router_logits)
    top_k_weights, selected_experts = jax.lax.top_k(router_probs, _TOPK)
    top_k_weights = top_k_weights / (top_k_weights.sum(axis=-1, keepdims=True) + 1e-20)
    onehot = (selected_experts[:, :, None] == jnp.arange(_E)[None, None, :])
    combine = jnp.sum(jnp.where(onehot, top_k_weights[:, :, None], 0.0), axis=1)

    combine_r = combine.T.reshape(_E, 1, T)
    w1_r = w1.reshape(d, _E, _W).transpose(1, 0, 2).astype(jnp.bfloat16)
    w2_r = w2.reshape(_E, _W, d).astype(jnp.bfloat16)

    out_flat = pl.pallas_call(
        _moe_body,
        grid=(_E,),
        in_specs=[
            pl.BlockSpec((T, d), lambda e: (0, 0)),
            pl.BlockSpec((1, d, _W), lambda e: (e, 0, 0)),
            pl.BlockSpec((1, _W, d), lambda e: (e, 0, 0)),
            pl.BlockSpec((1, 1, T), lambda e: (e, 0, 0)),
        ],
        out_specs=pl.BlockSpec((T, d), lambda e: (0, 0)),
        out_shape=jax.ShapeDtypeStruct((T, d), jnp.float32),
        compiler_params=pltpu.CompilerParams(
            dimension_semantics=("arbitrary",)),
    )(x_flat.astype(jnp.bfloat16), w1_r, w2_r, combine_r)

    output = out_flat.reshape(b, s, d)

    router_z_loss = jnp.mean(jnp.square(jax.nn.logsumexp(router_logits, axis=-1)))
    tokens_per_expert = jnp.sum(onehot, axis=(0, 1)).astype(jnp.float32)
    f_i = tokens_per_expert / (T * _TOPK)
    p_i = router_probs.mean(axis=0)
    load_balance_loss = float(_E) * (f_i @ p_i)
    compute_loss = jnp.mean(router_probs @ jnp.ones((_E,), jnp.float32))
    return (output, router_z_loss, load_balance_loss, compute_loss, f_i)
